# baseline (speedup 1.0000x reference)
import functools

import jax
import jax.numpy as jnp
from jax import lax
from jax.experimental import pallas as pl
from jax.experimental.pallas import tpu as pltpu
from jax.experimental.pallas import tpu_sc as plsc

D = 64
B = 4096
T = 200
BLK = 128
NBUF = 4
TGRP = 8
NC, NS = 2, 16
NW = NC * NS
LANES = 16


@jax.jit
def _sc_add_gather_t(xt, idxt, pe2):
    mesh = plsc.VectorSubcoreMesh(
        core_axis_name="c", subcore_axis_name="s",
        num_cores=NC, num_subcores=NS,
    )

    @functools.partial(
        pl.kernel,
        out_type=jax.ShapeDtypeStruct((T, D, B), jnp.float32),
        mesh=mesh,
        scratch_types=[
            pltpu.VMEM((T, BLK), jnp.int32),
            pltpu.VMEM((NBUF, BLK), jnp.int32),
            pltpu.VMEM((NBUF, BLK, BLK), jnp.float32),
            pltpu.VMEM((NBUF, D, BLK), jnp.float32),
            pltpu.VMEM((BLK // LANES, LANES), jnp.int32),
            pltpu.SemaphoreType.DMA((NBUF,)),
            pltpu.SemaphoreType.DMA((NBUF,)),
            pltpu.SemaphoreType.DMA((NBUF,)),
        ],
        compiler_params=pltpu.CompilerParams(
            use_tc_tiling_on_sc=True, needs_layout_passes=False),
    )
    def body(xt_hbm, idx_hbm, pe_hbm, out_hbm, idx_v, kidx_v, rows_v, x_v,
             cb_v, xsem, gsem, osem):
        wid = lax.axis_index("s") * NC + lax.axis_index("c")
        b0 = wid * BLK
        pltpu.sync_copy(idx_hbm.at[:, pl.ds(b0, BLK)], idx_v)

        iota = lax.iota(jnp.int32, LANES)
        iotas = [iota + (gi * LANES) for gi in range(BLK // LANES)]
        rots = [lax.bitwise_and(iota + k, LANES - 1) for k in range(LANES)]

        def in_copies(g, s):
            return (
                pltpu.make_async_copy(
                    xt_hbm.at[g, :, pl.ds(b0, BLK)], x_v.at[s], xsem.at[s]),
                pltpu.make_async_copy(
                    pe_hbm.at[kidx_v.at[s]], rows_v.at[s], gsem.at[s]),
            )

        def issue_in(g, s):
            for gi in range(BLK // LANES):
                iv = idx_v[g, pl.ds(gi * LANES, LANES)]
                kidx_v[s, pl.ds(gi * LANES, LANES)] = lax.shift_right_logical(
                    iv, 1)
            for c in in_copies(g, s):
                c.start()

        def out_copy(g, s):
            return pltpu.make_async_copy(
                x_v.at[s], out_hbm.at[g, :, pl.ds(b0, BLK)], osem.at[s])

        issue_in(0, 0)
        issue_in(1, 1)

        @pl.loop(0, T, step=TGRP)
        def _ring(g0):
            for bb in range(TGRP):
                s = bb % NBUF
                g = g0 + bb
                for c in in_copies(g, s):
                    c.wait()

                for gi in range(BLK // LANES):
                    iv = idx_v[g, pl.ds(gi * LANES, LANES)]
                    cb_v[gi] = lax.shift_left(lax.bitwise_and(iv, 1), 6)

                @pl.loop(0, (BLK // LANES) * (D // LANES), unroll=2)
                def _blk(j):
                    gi16 = lax.shift_left(lax.shift_right_logical(j, 2), 4)
                    dq16 = lax.shift_left(lax.bitwise_and(j, 3), 4)
                    rowv = iota + gi16
                    base = cb_v[lax.shift_right_logical(j, 2)] + dq16
                    for h in range(2):
                        ks = range(h * (LANES // 2), (h + 1) * (LANES // 2))
                        vals = [
                            plsc.load_gather(
                                rows_v.at[s], [rowv, rots[k] + base])
                            for k in ks
                        ]
                        for val, k in zip(vals, ks):
                            plsc.addupdate_scatter(
                                x_v.at[s], [rots[k] + dq16, rowv], val)

                out_copy(g, s).start()

                s2 = (bb + 2) % NBUF
                g2 = g + 2

                @pl.when(g2 < T)
                def _():
                    @pl.when(g2 >= NBUF)
                    def _():
                        out_copy(g2 - NBUF, s2).wait()

                    issue_in(g2, s2)

        out_copy(T - 2, (T - 2) % NBUF).wait()
        out_copy(T - 1, (T - 1) % NBUF).wait()

    return body(xt, idxt, pe2)


def kernel(x, indices, pe):
    xt = jnp.transpose(x, (1, 2, 0))
    idxt = jnp.transpose(indices, (1, 0))
    pe2 = pe.reshape(pe.shape[0] // 2, 2 * pe.shape[1])
    out_t = _sc_add_gather_t(xt, idxt, pe2)
    return jnp.transpose(out_t, (2, 0, 1))


# speedup vs baseline: 1.0028x; 1.0028x over previous
import functools

import jax
import jax.numpy as jnp
from jax import lax
from jax.experimental import pallas as pl
from jax.experimental.pallas import tpu as pltpu
from jax.experimental.pallas import tpu_sc as plsc

D = 64
B = 4096
T = 200
BLK = 128
NBUF = 4
TGRP = 8
NC, NS = 2, 16
NW = NC * NS
LANES = 16


@jax.jit
def _sc_add_gather_t(xt, idxt, pe2):
    mesh = plsc.VectorSubcoreMesh(
        core_axis_name="c", subcore_axis_name="s",
        num_cores=NC, num_subcores=NS,
    )

    @functools.partial(
        pl.kernel,
        out_type=jax.ShapeDtypeStruct((T, D, B), jnp.float32),
        mesh=mesh,
        scratch_types=[
            pltpu.VMEM((T, BLK), jnp.int32),
            pltpu.VMEM((NBUF, BLK), jnp.int32),
            pltpu.VMEM((NBUF, BLK, BLK), jnp.float32),
            pltpu.VMEM((NBUF, D, BLK), jnp.float32),
            pltpu.VMEM((BLK // LANES, LANES), jnp.int32),
            pltpu.SemaphoreType.DMA((NBUF,)),
            pltpu.SemaphoreType.DMA((NBUF,)),
            pltpu.SemaphoreType.DMA((NBUF,)),
        ],
        compiler_params=pltpu.CompilerParams(
            use_tc_tiling_on_sc=True, needs_layout_passes=False),
    )
    def body(xt_hbm, idx_hbm, pe_hbm, out_hbm, idx_v, kidx_v, rows_v, x_v,
             cb_v, xsem, gsem, osem):
        wid = lax.axis_index("s") * NC + lax.axis_index("c")
        b0 = wid * BLK
        pltpu.sync_copy(idx_hbm.at[:, pl.ds(b0, BLK)], idx_v)

        iota = lax.iota(jnp.int32, LANES)
        iotas = [iota + (gi * LANES) for gi in range(BLK // LANES)]
        rots = [lax.bitwise_and(iota + k, LANES - 1) for k in range(LANES)]

        def in_copies(g, s):
            return (
                pltpu.make_async_copy(
                    xt_hbm.at[g, :, pl.ds(b0, BLK)], x_v.at[s], xsem.at[s]),
                pltpu.make_async_copy(
                    pe_hbm.at[kidx_v.at[s]], rows_v.at[s], gsem.at[s]),
            )

        def issue_in(g, s):
            for gi in range(BLK // LANES):
                iv = idx_v[g, pl.ds(gi * LANES, LANES)]
                kidx_v[s, pl.ds(gi * LANES, LANES)] = lax.shift_right_logical(
                    iv, 1)
            for c in in_copies(g, s):
                c.start()

        def out_copy(g, s):
            return pltpu.make_async_copy(
                x_v.at[s], out_hbm.at[g, :, pl.ds(b0, BLK)], osem.at[s])

        issue_in(0, 0)
        issue_in(1, 1)

        @pl.loop(0, T, step=TGRP)
        def _ring(g0):
            for bb in range(TGRP):
                s = bb % NBUF
                g = g0 + bb
                for c in in_copies(g, s):
                    c.wait()

                for gi in range(BLK // LANES):
                    iv = idx_v[g, pl.ds(gi * LANES, LANES)]
                    cb_v[gi] = lax.shift_left(lax.bitwise_and(iv, 1), 6)

                @pl.loop(0, (BLK // LANES) * (D // LANES))
                def _blk(j):
                    gi16 = lax.shift_left(lax.shift_right_logical(j, 2), 4)
                    dq16 = lax.shift_left(lax.bitwise_and(j, 3), 4)
                    rowv = iota + gi16
                    base = cb_v[lax.shift_right_logical(j, 2)] + dq16
                    for h in range(2):
                        ks = range(h * (LANES // 2), (h + 1) * (LANES // 2))
                        vals = [
                            plsc.load_gather(
                                rows_v.at[s], [rowv, rots[k] + base])
                            for k in ks
                        ]
                        for val, k in zip(vals, ks):
                            plsc.addupdate_scatter(
                                x_v.at[s], [rots[k] + dq16, rowv], val)

                out_copy(g, s).start()

                s2 = (bb + 2) % NBUF
                g2 = g + 2

                @pl.when(g2 < T)
                def _():
                    @pl.when(g2 >= NBUF)
                    def _():
                        out_copy(g2 - NBUF, s2).wait()

                    issue_in(g2, s2)

        out_copy(T - 2, (T - 2) % NBUF).wait()
        out_copy(T - 1, (T - 1) % NBUF).wait()

    return body(xt, idxt, pe2)


def kernel(x, indices, pe):
    xt = jnp.transpose(x, (1, 2, 0))
    idxt = jnp.transpose(indices, (1, 0))
    pe2 = pe.reshape(pe.shape[0] // 2, 2 * pe.shape[1])
    out_t = _sc_add_gather_t(xt, idxt, pe2)
    return jnp.transpose(out_t, (2, 0, 1))


# speedup vs baseline: 1.0573x; 1.0544x over previous
import functools

import jax
import jax.numpy as jnp
from jax import lax
from jax.experimental import pallas as pl
from jax.experimental.pallas import tpu as pltpu
from jax.experimental.pallas import tpu_sc as plsc

D = 64
B = 4096
T = 200
BLK = 128
NBUF = 4
TGRP = 8
NC, NS = 2, 16
NW = NC * NS
LANES = 16


@jax.jit
def _sc_add_gather_t(xt, idxt, pe2):
    mesh = plsc.VectorSubcoreMesh(
        core_axis_name="c", subcore_axis_name="s",
        num_cores=NC, num_subcores=NS,
    )

    @functools.partial(
        pl.kernel,
        out_type=jax.ShapeDtypeStruct((T, D, B), jnp.float32),
        mesh=mesh,
        scratch_types=[
            pltpu.VMEM((T, BLK), jnp.int32),
            pltpu.VMEM((NBUF, BLK), jnp.int32),
            pltpu.VMEM((NBUF, BLK, BLK), jnp.float32),
            pltpu.VMEM((NBUF, D, BLK), jnp.float32),
            pltpu.VMEM((BLK // LANES, LANES), jnp.int32),
            pltpu.SemaphoreType.DMA((NBUF,)),
            pltpu.SemaphoreType.DMA((NBUF,)),
            pltpu.SemaphoreType.DMA((NBUF,)),
        ],
        compiler_params=pltpu.CompilerParams(
            use_tc_tiling_on_sc=True, needs_layout_passes=False),
    )
    def body(xt_hbm, idx_hbm, pe_hbm, out_hbm, idx_v, kidx_v, rows_v, x_v,
             cb_v, xsem, gsem, osem):
        wid = lax.axis_index("s") * NC + lax.axis_index("c")
        b0 = wid * BLK
        pltpu.sync_copy(idx_hbm.at[:, pl.ds(b0, BLK)], idx_v)

        iota = lax.iota(jnp.int32, LANES)
        iotas = [iota + (gi * LANES) for gi in range(BLK // LANES)]
        rots = [lax.bitwise_and(iota + k, LANES - 1) for k in range(LANES)]

        def in_copies(g, s):
            return (
                pltpu.make_async_copy(
                    xt_hbm.at[g, :, pl.ds(b0, BLK)], x_v.at[s], xsem.at[s]),
                pltpu.make_async_copy(
                    pe_hbm.at[kidx_v.at[s]], rows_v.at[s], gsem.at[s]),
            )

        def issue_in(g, s):
            for gi in range(BLK // LANES):
                iv = idx_v[g, pl.ds(gi * LANES, LANES)]
                kidx_v[s, pl.ds(gi * LANES, LANES)] = lax.shift_right_logical(
                    iv, 1)
            for c in in_copies(g, s):
                c.start()

        def out_copy(g, s):
            return pltpu.make_async_copy(
                x_v.at[s], out_hbm.at[g, :, pl.ds(b0, BLK)], osem.at[s])

        issue_in(0, 0)
        issue_in(1, 1)

        @pl.loop(0, T, step=TGRP)
        def _ring(g0):
            for bb in range(TGRP):
                s = bb % NBUF
                g = g0 + bb
                for c in in_copies(g, s):
                    c.wait()

                s2 = (bb + 2) % NBUF
                g2 = g + 2

                @pl.when(g2 < T)
                def _():
                    @pl.when(g2 >= NBUF)
                    def _():
                        out_copy(g2 - NBUF, s2).wait()

                    issue_in(g2, s2)

                for gi in range(BLK // LANES):
                    iv = idx_v[g, pl.ds(gi * LANES, LANES)]
                    cb_v[gi] = lax.shift_left(lax.bitwise_and(iv, 1), 6)

                @pl.loop(0, (BLK // LANES) * (D // LANES))
                def _blk(j):
                    gi16 = lax.shift_left(lax.shift_right_logical(j, 2), 4)
                    dq16 = lax.shift_left(lax.bitwise_and(j, 3), 4)
                    rowv = iota + gi16
                    base = cb_v[lax.shift_right_logical(j, 2)] + dq16
                    for h in range(2):
                        ks = range(h * (LANES // 2), (h + 1) * (LANES // 2))
                        vals = [
                            plsc.load_gather(
                                rows_v.at[s], [rowv, rots[k] + base])
                            for k in ks
                        ]
                        for val, k in zip(vals, ks):
                            plsc.addupdate_scatter(
                                x_v.at[s], [rots[k] + dq16, rowv], val)

                out_copy(g, s).start()

        out_copy(T - 2, (T - 2) % NBUF).wait()
        out_copy(T - 1, (T - 1) % NBUF).wait()

    return body(xt, idxt, pe2)


def kernel(x, indices, pe):
    xt = jnp.transpose(x, (1, 2, 0))
    idxt = jnp.transpose(indices, (1, 0))
    pe2 = pe.reshape(pe.shape[0] // 2, 2 * pe.shape[1])
    out_t = _sc_add_gather_t(xt, idxt, pe2)
    return jnp.transpose(out_t, (2, 0, 1))


# speedup vs baseline: 1.0583x; 1.0009x over previous
import functools

import jax
import jax.numpy as jnp
from jax import lax
from jax.experimental import pallas as pl
from jax.experimental.pallas import tpu as pltpu
from jax.experimental.pallas import tpu_sc as plsc

D = 64
B = 4096
T = 200
BLK = 128
NBUF = 4
TGRP = 8
NC, NS = 2, 16
NW = NC * NS
LANES = 16


@jax.jit
def _sc_add_gather_t(xt, idxt, pe2):
    mesh = plsc.VectorSubcoreMesh(
        core_axis_name="c", subcore_axis_name="s",
        num_cores=NC, num_subcores=NS,
    )

    @functools.partial(
        pl.kernel,
        out_type=jax.ShapeDtypeStruct((T, D, B), jnp.float32),
        mesh=mesh,
        scratch_types=[
            pltpu.VMEM((T, BLK), jnp.int32),
            pltpu.VMEM((NBUF, BLK), jnp.int32),
            pltpu.VMEM((NBUF, BLK, BLK), jnp.float32),
            pltpu.VMEM((NBUF, D, BLK), jnp.float32),
            pltpu.VMEM((BLK // LANES, LANES), jnp.int32),
            pltpu.SemaphoreType.DMA((NBUF,)),
            pltpu.SemaphoreType.DMA((NBUF,)),
            pltpu.SemaphoreType.DMA((NBUF,)),
        ],
        compiler_params=pltpu.CompilerParams(
            use_tc_tiling_on_sc=True, needs_layout_passes=False),
    )
    def body(xt_hbm, idx_hbm, pe_hbm, out_hbm, idx_v, kidx_v, rows_v, x_v,
             cb_v, xsem, gsem, osem):
        wid = lax.axis_index("s") * NC + lax.axis_index("c")
        b0 = wid * BLK
        pltpu.sync_copy(idx_hbm.at[:, pl.ds(b0, BLK)], idx_v)

        iota = lax.iota(jnp.int32, LANES)
        rots = [lax.bitwise_and(iota + k, LANES - 1) for k in range(LANES)]

        def in_copies(g, s):
            return (
                pltpu.make_async_copy(
                    xt_hbm.at[g, :, pl.ds(b0, BLK)], x_v.at[s], xsem.at[s]),
                pltpu.make_async_copy(
                    pe_hbm.at[kidx_v.at[s]], rows_v.at[s], gsem.at[s]),
            )

        def issue_in(g, s):
            for gi in range(BLK // LANES):
                iv = idx_v[g, pl.ds(gi * LANES, LANES)]
                kidx_v[s, pl.ds(gi * LANES, LANES)] = lax.shift_right_logical(
                    iv, 1)
            for c in in_copies(g, s):
                c.start()

        def out_copy(g, s):
            return pltpu.make_async_copy(
                x_v.at[s], out_hbm.at[g, :, pl.ds(b0, BLK)], osem.at[s])

        issue_in(0, 0)
        issue_in(1, 1)

        @pl.loop(0, T, step=TGRP)
        def _ring(g0):
            for bb in range(TGRP):
                s = bb % NBUF
                g = g0 + bb
                for c in in_copies(g, s):
                    c.wait()

                s2 = (bb + 2) % NBUF
                g2 = g + 2

                @pl.when(g2 < T)
                def _():
                    @pl.when(g2 >= NBUF)
                    def _():
                        out_copy(g2 - NBUF, s2).wait()

                    issue_in(g2, s2)

                for gi in range(BLK // LANES):
                    iv = idx_v[g, pl.ds(gi * LANES, LANES)]
                    cb_v[gi] = lax.shift_left(lax.bitwise_and(iv, 1), 6)

                @pl.loop(0, (BLK // LANES) * (D // LANES))
                def _blk(j):
                    gi16 = lax.shift_left(lax.shift_right_logical(j, 2), 4)
                    dq16 = lax.shift_left(lax.bitwise_and(j, 3), 4)
                    rowv = iota + gi16
                    base = cb_v[lax.shift_right_logical(j, 2)] + dq16
                    for h in range(2):
                        ks = range(h * (LANES // 2), (h + 1) * (LANES // 2))
                        vals = [
                            plsc.load_gather(
                                rows_v.at[s], [rowv, rots[k] + base])
                            for k in ks
                        ]
                        for val, k in zip(vals, ks):
                            plsc.addupdate_scatter(
                                x_v.at[s], [rots[k] + dq16, rowv], val)

                out_copy(g, s).start()

        out_copy(T - 2, (T - 2) % NBUF).wait()
        out_copy(T - 1, (T - 1) % NBUF).wait()

    return body(xt, idxt, pe2)


def kernel(x, indices, pe):
    xt = jnp.transpose(x, (1, 2, 0))
    idxt = jnp.transpose(indices, (1, 0))
    pe2 = pe.reshape(pe.shape[0] // 2, 2 * pe.shape[1])
    out_t = _sc_add_gather_t(xt, idxt, pe2)
    return jnp.transpose(out_t, (2, 0, 1))
